# baseline (device time: 114197 ns/iter reference)
import jax
import jax.numpy as jnp
from jax import lax
from jax.experimental import pallas as pl
from jax.experimental.pallas import tpu as pltpu

N_DEV = 4
K = 32
NEG = float("-inf")


def _topk_local(x):
    m, n = x.shape
    block_m = 128
    C = 128
    T = 6
    NB = n // C

    def body(x_ref, o_ref, s_ref):
        g = [jnp.full((block_m, C), NEG, jnp.bfloat16) for _ in range(T)]
        for b in range(NB):
            w = x_ref[:, b * C:(b + 1) * C].astype(jnp.bfloat16)
            for t in range(T):
                hi = jnp.maximum(g[t], w)
                w = jnp.minimum(g[t], w)
                g[t] = hi
        g = [gt.astype(jnp.float32) for gt in g]
        for t in range(T):
            s_ref[:, t * C:(t + 1) * C] = g[t]

        iota = lax.broadcasted_iota(jnp.int32, (block_m, K), 1)

        def jstep(j, carry):
            acc, m_prev = carry
            sv = s_ref[:, :]
            masked = jnp.where(sv < m_prev, sv, NEG)
            mx = jnp.max(masked, axis=1, keepdims=True)
            return jnp.where(iota == j, mx, acc), mx

        acc0 = jnp.full((block_m, K), NEG, jnp.float32)
        m0 = jnp.full((block_m, 1), float("inf"), jnp.float32)
        acc, m_last = lax.fori_loop(0, K, jstep, (acc0, m0))
        o_ref[:, :] = acc

        @pl.when(jnp.any(g[T - 1] >= m_last))
        def _():
            def kstep(i, carry):
                acc, m_prev = carry
                gm = jnp.full((block_m, C), NEG, jnp.float32)
                for b in range(NB):
                    blk = x_ref[:, b * C:(b + 1) * C]
                    gm = jnp.maximum(gm, jnp.where(blk < m_prev, blk, NEG))
                mx = jnp.max(gm, axis=1, keepdims=True)
                return jnp.where(iota == i, mx, acc), mx

            m0 = jnp.full((block_m, 1), float("inf"), jnp.float32)
            acc2, _ = lax.fori_loop(0, K, kstep, (acc0, m0))
            o_ref[:, :] = acc2

    return pl.pallas_call(
        body,
        grid=(m // block_m,),
        in_specs=[pl.BlockSpec((block_m, n), lambda i: (i, 0))],
        out_specs=pl.BlockSpec((block_m, K), lambda i: (i, 0)),
        out_shape=jax.ShapeDtypeStruct((m, K), jnp.float32),
        scratch_shapes=[pltpu.VMEM((block_m, T * C), jnp.float32)],
    )(x)


def _gather_merge(cand):
    m, _ = cand.shape

    def body(c_ref, o_ref, comm_ref, w_ref, send_sems, recv_sems):
        my = lax.axis_index("i")
        left = lax.rem(my - 1 + N_DEV, N_DEV)
        right = lax.rem(my + 1, N_DEV)

        barrier_sem = pltpu.get_barrier_semaphore()
        for nbr in (left, right):
            pl.semaphore_signal(
                barrier_sem,
                inc=1,
                device_id=(nbr,),
                device_id_type=pl.DeviceIdType.MESH,
            )
        pl.semaphore_wait(barrier_sem, 2)

        comm_ref[0] = c_ref[:, :]

        for h in range(N_DEV - 1):
            rdma = pltpu.make_async_remote_copy(
                src_ref=comm_ref.at[h],
                dst_ref=comm_ref.at[h + 1],
                send_sem=send_sems.at[h],
                recv_sem=recv_sems.at[h],
                device_id=(right,),
                device_id_type=pl.DeviceIdType.MESH,
            )
            rdma.start()
            rdma.wait()

        for d in range(N_DEV):
            w_ref[:, d * K:(d + 1) * K] = comm_ref[d]

        iota = lax.broadcasted_iota(jnp.int32, (m, K), 1)

        def kstep(i, acc):
            wv = w_ref[:, :]
            mx = jnp.max(wv, axis=1, keepdims=True)
            w_ref[:, :] = jnp.where(wv == mx, NEG, wv)
            return jnp.where(iota == i, mx, acc)

        acc0 = jnp.full((m, K), NEG, jnp.float32)
        o_ref[:, :] = lax.fori_loop(0, K, kstep, acc0)

    return pl.pallas_call(
        body,
        out_shape=jax.ShapeDtypeStruct((m, K), jnp.float32),
        in_specs=[pl.BlockSpec(memory_space=pltpu.VMEM)],
        out_specs=pl.BlockSpec(memory_space=pltpu.VMEM),
        scratch_shapes=[
            pltpu.VMEM((N_DEV, m, K), jnp.float32),
            pltpu.VMEM((m, N_DEV * K), jnp.float32),
            pltpu.SemaphoreType.DMA((N_DEV - 1,)),
            pltpu.SemaphoreType.DMA((N_DEV - 1,)),
        ],
        compiler_params=pltpu.CompilerParams(collective_id=0),
    )(cand)


def kernel(x):
    cand = _topk_local(x)
    return _gather_merge(cand)


# device time: 84637 ns/iter; 1.3493x vs baseline; 1.3493x over previous
import jax
import jax.numpy as jnp
from jax import lax
from jax.experimental import pallas as pl
from jax.experimental.pallas import tpu as pltpu

N_DEV = 4
K = 32
NEG = float("-inf")


def _topk_local(x):
    m, n = x.shape
    block_m = 128
    C = 128
    T = 6
    NB = n // C

    def body(x_ref, o_ref, s_ref):
        g = [jnp.full((block_m, C), NEG, jnp.float32) for _ in range(T)]
        for b in range(NB):
            w = x_ref[:, b * C:(b + 1) * C]
            for t in range(T):
                hi = jnp.maximum(g[t], w)
                w = jnp.minimum(g[t], w)
                g[t] = hi
        for t in range(T):
            s_ref[:, t * C:(t + 1) * C] = g[t]

        iota = lax.broadcasted_iota(jnp.int32, (block_m, K), 1)

        def jstep(j, carry):
            acc, m_prev = carry
            sv = s_ref[:, :]
            masked = jnp.where(sv < m_prev, sv, NEG)
            mx = jnp.max(masked, axis=1, keepdims=True)
            return jnp.where(iota == j, mx, acc), mx

        acc0 = jnp.full((block_m, K), NEG, jnp.float32)
        m0 = jnp.full((block_m, 1), float("inf"), jnp.float32)
        acc, m_last = lax.fori_loop(0, K, jstep, (acc0, m0))
        o_ref[:, :] = acc

        @pl.when(jnp.any(g[T - 1] >= m_last))
        def _():
            def kstep(i, carry):
                acc, m_prev = carry
                gm = jnp.full((block_m, C), NEG, jnp.float32)
                for b in range(NB):
                    blk = x_ref[:, b * C:(b + 1) * C]
                    gm = jnp.maximum(gm, jnp.where(blk < m_prev, blk, NEG))
                mx = jnp.max(gm, axis=1, keepdims=True)
                return jnp.where(iota == i, mx, acc), mx

            m0 = jnp.full((block_m, 1), float("inf"), jnp.float32)
            acc2, _ = lax.fori_loop(0, K, kstep, (acc0, m0))
            o_ref[:, :] = acc2

    return pl.pallas_call(
        body,
        grid=(m // block_m,),
        in_specs=[pl.BlockSpec((block_m, n), lambda i: (i, 0))],
        out_specs=pl.BlockSpec((block_m, K), lambda i: (i, 0)),
        out_shape=jax.ShapeDtypeStruct((m, K), jnp.float32),
        scratch_shapes=[pltpu.VMEM((block_m, T * C), jnp.float32)],
    )(x)


def _gather_merge(cand):
    m, _ = cand.shape

    def body(c_ref, o_ref, comm_ref, w_ref, send_sems, recv_sems):
        my = lax.axis_index("i")

        barrier_sem = pltpu.get_barrier_semaphore()
        for j in range(1, N_DEV):
            pl.semaphore_signal(
                barrier_sem,
                inc=1,
                device_id=(lax.rem(my + j, N_DEV),),
                device_id_type=pl.DeviceIdType.MESH,
            )
        pl.semaphore_wait(barrier_sem, N_DEV - 1)

        rdmas = []
        for j in range(1, N_DEV):
            rdma = pltpu.make_async_remote_copy(
                src_ref=c_ref,
                dst_ref=comm_ref.at[N_DEV - j],
                send_sem=send_sems.at[j - 1],
                recv_sem=recv_sems.at[N_DEV - j],
                device_id=(lax.rem(my + j, N_DEV),),
                device_id_type=pl.DeviceIdType.MESH,
            )
            rdma.start()
            rdmas.append(rdma)
        w_ref[:, 0:K] = c_ref[:, :]
        for j, rdma in enumerate(rdmas, start=1):
            rdma.wait()
            s = N_DEV - j
            w_ref[:, s * K:(s + 1) * K] = comm_ref[s]

        iota = lax.broadcasted_iota(jnp.int32, (m, K), 1)

        def kstep(i, carry):
            acc, m_prev = carry
            wv = w_ref[:, :]
            masked = jnp.where(wv < m_prev, wv, NEG)
            mx = jnp.max(masked, axis=1, keepdims=True)
            return jnp.where(iota == i, mx, acc), mx

        acc0 = jnp.full((m, K), NEG, jnp.float32)
        m0 = jnp.full((m, 1), float("inf"), jnp.float32)
        acc, _ = lax.fori_loop(0, K, kstep, (acc0, m0))
        o_ref[:, :] = acc

    return pl.pallas_call(
        body,
        out_shape=jax.ShapeDtypeStruct((m, K), jnp.float32),
        in_specs=[pl.BlockSpec(memory_space=pltpu.VMEM)],
        out_specs=pl.BlockSpec(memory_space=pltpu.VMEM),
        scratch_shapes=[
            pltpu.VMEM((N_DEV, m, K), jnp.float32),
            pltpu.VMEM((m, N_DEV * K), jnp.float32),
            pltpu.SemaphoreType.DMA((N_DEV - 1,)),
            pltpu.SemaphoreType.DMA((N_DEV,)),
        ],
        compiler_params=pltpu.CompilerParams(collective_id=0),
    )(cand)


def kernel(x):
    cand = _topk_local(x)
    return _gather_merge(cand)


# device time: 79879 ns/iter; 1.4296x vs baseline; 1.0596x over previous
import jax
import jax.numpy as jnp
from jax import lax
from jax.experimental import pallas as pl
from jax.experimental.pallas import tpu as pltpu

N_DEV = 4
K = 32
NEG = float("-inf")


def _topk_local(x):
    m, n = x.shape
    block_m = 128
    C = 128
    T = 6
    NB = n // C

    def body(x_ref, o_ref, s_ref):
        g = [jnp.full((block_m, C), NEG, jnp.float32) for _ in range(T)]
        for b in range(NB):
            w = x_ref[:, b * C:(b + 1) * C]
            for t in range(T):
                hi = jnp.maximum(g[t], w)
                w = jnp.minimum(g[t], w)
                g[t] = hi
        for t in range(T):
            s_ref[:, t * C:(t + 1) * C] = g[t]

        iota = lax.broadcasted_iota(jnp.int32, (block_m, K), 1)

        R = 2

        def jstep(j, carry):
            acc, m_prev = carry
            masked = s_ref[:, :]
            for r in range(R):
                masked = jnp.where(masked < m_prev, masked, NEG)
                m_prev = jnp.max(masked, axis=1, keepdims=True)
                acc = jnp.where(iota == R * j + r, m_prev, acc)
            return acc, m_prev

        acc0 = jnp.full((block_m, K), NEG, jnp.float32)
        m0 = jnp.full((block_m, 1), float("inf"), jnp.float32)
        acc, m_last = lax.fori_loop(0, K // R, jstep, (acc0, m0))
        o_ref[:, :] = acc

        @pl.when(jnp.any(g[T - 1] >= m_last))
        def _():
            def kstep(i, carry):
                acc, m_prev = carry
                gm = jnp.full((block_m, C), NEG, jnp.float32)
                for b in range(NB):
                    blk = x_ref[:, b * C:(b + 1) * C]
                    gm = jnp.maximum(gm, jnp.where(blk < m_prev, blk, NEG))
                mx = jnp.max(gm, axis=1, keepdims=True)
                return jnp.where(iota == i, mx, acc), mx

            m0 = jnp.full((block_m, 1), float("inf"), jnp.float32)
            acc2, _ = lax.fori_loop(0, K, kstep, (acc0, m0))
            o_ref[:, :] = acc2

    return pl.pallas_call(
        body,
        grid=(m // block_m,),
        in_specs=[pl.BlockSpec((block_m, n), lambda i: (i, 0))],
        out_specs=pl.BlockSpec((block_m, K), lambda i: (i, 0)),
        out_shape=jax.ShapeDtypeStruct((m, K), jnp.float32),
        scratch_shapes=[pltpu.VMEM((block_m, T * C), jnp.float32)],
    )(x)


def _gather_merge(cand):
    m, _ = cand.shape

    def body(c_ref, o_ref, comm_ref, w_ref, send_sems, recv_sems):
        my = lax.axis_index("i")

        barrier_sem = pltpu.get_barrier_semaphore()
        for j in range(1, N_DEV):
            pl.semaphore_signal(
                barrier_sem,
                inc=1,
                device_id=(lax.rem(my + j, N_DEV),),
                device_id_type=pl.DeviceIdType.MESH,
            )
        pl.semaphore_wait(barrier_sem, N_DEV - 1)

        rdmas = []
        for j in range(1, N_DEV):
            rdma = pltpu.make_async_remote_copy(
                src_ref=c_ref,
                dst_ref=comm_ref.at[N_DEV - j],
                send_sem=send_sems.at[j - 1],
                recv_sem=recv_sems.at[N_DEV - j],
                device_id=(lax.rem(my + j, N_DEV),),
                device_id_type=pl.DeviceIdType.MESH,
            )
            rdma.start()
            rdmas.append(rdma)
        w_ref[:, 0:K] = c_ref[:, :]
        for j, rdma in enumerate(rdmas, start=1):
            rdma.wait()
            s = N_DEV - j
            w_ref[:, s * K:(s + 1) * K] = comm_ref[s]

        iota = lax.broadcasted_iota(jnp.int32, (m, K), 1)

        def kstep(i, carry):
            acc, m_prev = carry
            wv = w_ref[:, :]
            masked = jnp.where(wv < m_prev, wv, NEG)
            mx = jnp.max(masked, axis=1, keepdims=True)
            return jnp.where(iota == i, mx, acc), mx

        acc0 = jnp.full((m, K), NEG, jnp.float32)
        m0 = jnp.full((m, 1), float("inf"), jnp.float32)
        acc, _ = lax.fori_loop(0, K, kstep, (acc0, m0))
        o_ref[:, :] = acc

    return pl.pallas_call(
        body,
        out_shape=jax.ShapeDtypeStruct((m, K), jnp.float32),
        in_specs=[pl.BlockSpec(memory_space=pltpu.VMEM)],
        out_specs=pl.BlockSpec(memory_space=pltpu.VMEM),
        scratch_shapes=[
            pltpu.VMEM((N_DEV, m, K), jnp.float32),
            pltpu.VMEM((m, N_DEV * K), jnp.float32),
            pltpu.SemaphoreType.DMA((N_DEV - 1,)),
            pltpu.SemaphoreType.DMA((N_DEV,)),
        ],
        compiler_params=pltpu.CompilerParams(collective_id=0),
    )(cand)


def kernel(x):
    cand = _topk_local(x)
    return _gather_merge(cand)
